# baseline (speedup 1.0000x reference)
import functools

import jax
import jax.numpy as jnp
from jax import lax
from jax.experimental import pallas as pl
from jax.experimental.pallas import tpu as pltpu
from jax.experimental.pallas import tpu_sc as plsc

N = 10000
E = 160000
D = 128
H = 2
C = 768
G = 8

NT = 32
NP = 10240
EP = 160256
CE = EP // NT
R1 = NP // NT
RN = 80
NPASS = NP // (NT * RN)
NB = NP // 256
EB = EP // 512
F32 = jnp.float32
I32 = jnp.int32


def _tc_pre_body(x_ref, lw_ref, atts_ref, attd_ref, xh_ref, a8_ref, w_scr):
    i = pl.program_id(0)

    @pl.when(i == 0)
    def _():
        lw = lw_ref[...]
        asf = atts_ref[...]
        adf = attd_ref[...]
        parts = []
        for tab in (asf, adf):
            for h in range(H):
                parts.append(lax.dot_general(
                    tab[h:h + 1, :], lw[:, h * C:(h + 1) * C],
                    (((1,), (1,)), ((), ())), preferred_element_type=F32))
        parts.append(jnp.zeros((4, 128), F32))
        w_scr[...] = jnp.concatenate(parts, axis=0)

    xb = x_ref[...]
    xh_ref[...] = lax.dot_general(xb, lw_ref[...], (((1,), (0,)), ((), ())),
                                  preferred_element_type=F32)
    a8_ref[...] = lax.dot_general(w_scr[...], xb, (((1,), (1,)), ((), ())),
                                  preferred_element_type=F32)


def _tc_ae_body(e_ref, lw_ref, atte_ref, ae8_ref, w_scr):
    i = pl.program_id(0)

    @pl.when(i == 0)
    def _():
        lw = lw_ref[...]
        aef = atte_ref[...]
        parts = []
        for h in range(H):
            parts.append(lax.dot_general(
                aef[h:h + 1, :], lw[:, h * C:(h + 1) * C],
                (((1,), (1,)), ((), ())), preferred_element_type=F32))
        parts.append(jnp.zeros((6, 128), F32))
        w_scr[...] = jnp.concatenate(parts, axis=0)

    ae8_ref[...] = lax.dot_general(w_scr[...], e_ref[...],
                                   (((1,), (1,)), ((), ())),
                                   preferred_element_type=F32)


def _onehot_t(b3_ref):
    bv = b3_ref[0]
    iota8 = lax.broadcasted_iota(I32, (G, 1), 0)
    return (bv == iota8).astype(F32)


def _tc_b1_body(agg_ref, bias_ref, b3_ref, s1_ref, s2_ref, cnt_ref):
    i = pl.program_id(0)

    @pl.when(i == 0)
    def _():
        s1_ref[...] = jnp.zeros((G, C), F32)
        s2_ref[...] = jnp.zeros((G, C), F32)
        cnt_ref[...] = jnp.zeros((G, 128), F32)

    hp = 0.5 * agg_ref[...] + bias_ref[...]
    obt = _onehot_t(b3_ref)
    s1_ref[...] += lax.dot_general(obt, hp, (((1,), (0,)), ((), ())),
                                   preferred_element_type=F32)
    s2_ref[...] += lax.dot_general(obt, hp * hp, (((1,), (0,)), ((), ())),
                                   preferred_element_type=F32)
    cnt_ref[...] += lax.dot_general(obt, jnp.ones((256, 128), F32),
                                    (((1,), (0,)), ((), ())),
                                    preferred_element_type=F32)


def _tc_b2_body(agg_ref, bias_ref, b3_ref, x_ref, s1_ref, s2_ref, cnt_ref,
                gnw_ref, gnb_ref, gms_ref, resw_ref, resb_ref, pa_ref,
                g1w_ref, g1b_ref, pa1_ref, g2w_ref, g2b_ref, pa2_ref,
                g3w_ref, g3b_ref,
                h3_ref, gate3_ref, gmax_ref, a_scr, b_scr):
    i = pl.program_id(0)

    @pl.when(i == 0)
    def _():
        c = jnp.maximum(cnt_ref[...][:, 0:1], 1.0)
        s1 = s1_ref[...]
        mean = s1 / c
        mt = mean * gms_ref[...]
        var = s2_ref[...] / c - 2.0 * mt * (s1 / c) + mt * mt
        istd = lax.rsqrt(var + 1e-5)
        av = gnw_ref[...] * istd
        a_scr[...] = av
        b_scr[...] = gnb_ref[...] - mt * av
        gmax_ref[...] = jnp.full((G, 128), -1e30, F32)

    hp = 0.5 * agg_ref[...] + bias_ref[...]
    obt = _onehot_t(b3_ref)
    hn = hp * lax.dot_general(obt, a_scr[...], (((0,), (0,)), ((), ())),
                              preferred_element_type=F32)
    hn = hn + lax.dot_general(obt, b_scr[...], (((0,), (0,)), ((), ())),
                              preferred_element_type=F32)
    h3 = hn + lax.dot_general(x_ref[...], resw_ref[...],
                              (((1,), (0,)), ((), ())),
                              preferred_element_type=F32) + resb_ref[...]
    h3 = jnp.where(h3 >= 0.0, h3, pa_ref[...] * h3)
    h3_ref[...] = h3
    g = lax.dot_general(h3, g1w_ref[...], (((1,), (0,)), ((), ())),
                        preferred_element_type=F32) + g1b_ref[...]
    g = jnp.where(g >= 0.0, g, pa1_ref[...] * g)
    g = lax.dot_general(g, g2w_ref[...], (((1,), (0,)), ((), ())),
                        preferred_element_type=F32) + g2b_ref[...]
    g = jnp.where(g >= 0.0, g, pa2_ref[...] * g)
    gate_row = lax.dot_general(g3w_ref[...], g, (((0,), (1,)), ((), ())),
                               preferred_element_type=F32) + g3b_ref[...]
    gate3_ref[...] = gate_row.reshape(1, 1, 256)
    mg = jnp.where(obt > 0.0, gate_row, -1e30)
    gm = jnp.max(mg, axis=1, keepdims=True)
    gmax_ref[...] = jnp.maximum(gmax_ref[...], jnp.broadcast_to(gm, (G, 128)))


def _tc_b3_body(h3_ref, gate3_ref, gmax_ref, b3_ref, pooled_ref,
                sgeh_scr, gden_scr):
    i = pl.program_id(0)

    @pl.when(i == 0)
    def _():
        sgeh_scr[...] = jnp.zeros((G, C), F32)
        gden_scr[...] = jnp.zeros((G, 128), F32)

    obt = _onehot_t(b3_ref)
    gmv = gmax_ref[...][:, 0:1]
    gmv = jnp.where(gmv > -1e29, gmv, 0.0)
    gm_row = lax.dot_general(gmv, obt, (((0,), (0,)), ((), ())),
                             preferred_element_type=F32)
    ge_row = jnp.exp(gate3_ref[0] - gm_row)
    wob = obt * ge_row
    sgeh_scr[...] += lax.dot_general(wob, h3_ref[...], (((1,), (0,)), ((), ())),
                                     preferred_element_type=F32)
    gden_scr[...] += lax.dot_general(wob, jnp.ones((256, 128), F32),
                                     (((1,), (0,)), ((), ())),
                                     preferred_element_type=F32)
    pooled_ref[...] = sgeh_scr[...] / (gden_scr[...][:, 0:1] + 1e-16)


def _lrelu(v):
    return jnp.where(v >= 0.0, v, 0.2 * v)


def _wid():
    return lax.axis_index("c") * 16 + lax.axis_index("s")


def _zero_arr(ref, ngroups):
    z = jnp.zeros((16,), F32)

    def b(i, _):
        ref[pl.ds(i * 16, 16)] = z
        return 0
    lax.fori_loop(0, ngroups, b, 0)


def _k1_body(a4f, aef, srcm, dstm, denf, swf,
             as0, as1, ad0, ad1,
             d0a, d1a, sa0a, sa1a, dga,
             src_v, dst_v, ae0_v, ae1_v,
             den0_o, den1_o, sw0_o, sw1_o):
    wid = _wid()
    lo = wid * R1
    pltpu.sync_copy(a4f.at[pl.ds(0 * NP, NP)], as0)
    pltpu.sync_copy(a4f.at[pl.ds(1 * NP, NP)], as1)
    pltpu.sync_copy(a4f.at[pl.ds(2 * NP, NP)], ad0)
    pltpu.sync_copy(a4f.at[pl.ds(3 * NP, NP)], ad1)
    lane = lax.iota(I32, 16)
    for r in (d0a, d1a, sa0a, sa1a, dga):
        _zero_arr(r, R1)
    ones16 = jnp.ones((16,), F32)

    def chunk(ci, _):
        base = ci * CE
        pltpu.sync_copy(srcm.at[pl.ds(base, CE)], src_v)
        pltpu.sync_copy(dstm.at[pl.ds(base, CE)], dst_v)
        pltpu.sync_copy(aef.at[pl.ds(base, CE)], ae0_v)
        pltpu.sync_copy(aef.at[pl.ds(EP + base, CE)], ae1_v)

        def grp(gi, _):
            s = src_v[pl.ds(gi * 16, 16)]
            d = dst_v[pl.ds(gi * 16, 16)]
            e0 = ae0_v[pl.ds(gi * 16, 16)]
            e1 = ae1_v[pl.ds(gi * 16, 16)]
            x0 = jnp.exp(_lrelu(plsc.load_gather(as0, [s])
                                + plsc.load_gather(ad0, [d]) + e0))
            x1 = jnp.exp(_lrelu(plsc.load_gather(as1, [s])
                                + plsc.load_gather(ad1, [d]) + e1))
            dl = d - lo
            m = (dl >= 0) & (dl < R1)
            idx = lane * R1 + jnp.where(m, dl, 0)
            plsc.addupdate_scatter(d0a, [idx], x0, mask=m)
            plsc.addupdate_scatter(d1a, [idx], x1, mask=m)
            plsc.addupdate_scatter(sa0a, [idx], e0, mask=m)
            plsc.addupdate_scatter(sa1a, [idx], e1, mask=m)
            plsc.addupdate_scatter(dga, [idx], ones16, mask=m)
            return 0

        lax.fori_loop(0, CE // 16, grp, 0)
        return 0

    lax.fori_loop(0, NT, chunk, 0)

    def fin(gi, _):
        off = gi * 16

        def red(ref):
            v = jnp.zeros((16,), F32)
            for l in range(16):
                v = v + ref[pl.ds(l * R1 + off, 16)]
            return v

        den0 = red(d0a)
        den1 = red(d1a)
        s0 = red(sa0a)
        s1v = red(sa1a)
        dg = jnp.maximum(red(dga), 1.0)
        el0 = jnp.exp(_lrelu(as0[pl.ds(lo + off, 16)]
                             + ad0[pl.ds(lo + off, 16)] + s0 / dg))
        el1 = jnp.exp(_lrelu(as1[pl.ds(lo + off, 16)]
                             + ad1[pl.ds(lo + off, 16)] + s1v / dg))
        t0 = den0 + el0
        t1 = den1 + el1
        den0_o[pl.ds(off, 16)] = t0
        den1_o[pl.ds(off, 16)] = t1
        sw0_o[pl.ds(off, 16)] = el0 / (t0 + 1e-16)
        sw1_o[pl.ds(off, 16)] = el1 / (t1 + 1e-16)
        return 0

    lax.fori_loop(0, R1 // 16, fin, 0)
    pltpu.sync_copy(den0_o, denf.at[pl.ds(lo, R1)])
    pltpu.sync_copy(den1_o, denf.at[pl.ds(NP + lo, R1)])
    pltpu.sync_copy(sw0_o, swf.at[pl.ds(lo, R1)])
    pltpu.sync_copy(sw1_o, swf.at[pl.ds(NP + lo, R1)])


def _k1b_body(a4f, aef, denf, srcm, dstm, wf,
              as0, as1, ad0, ad1, dn0, dn1,
              src_v, dst_v, ae0_v, ae1_v, w0_v, w1_v):
    wid = _wid()
    base = wid * CE
    pltpu.sync_copy(a4f.at[pl.ds(0 * NP, NP)], as0)
    pltpu.sync_copy(a4f.at[pl.ds(1 * NP, NP)], as1)
    pltpu.sync_copy(a4f.at[pl.ds(2 * NP, NP)], ad0)
    pltpu.sync_copy(a4f.at[pl.ds(3 * NP, NP)], ad1)
    pltpu.sync_copy(denf.at[pl.ds(0 * NP, NP)], dn0)
    pltpu.sync_copy(denf.at[pl.ds(1 * NP, NP)], dn1)
    pltpu.sync_copy(srcm.at[pl.ds(base, CE)], src_v)
    pltpu.sync_copy(dstm.at[pl.ds(base, CE)], dst_v)
    pltpu.sync_copy(aef.at[pl.ds(base, CE)], ae0_v)
    pltpu.sync_copy(aef.at[pl.ds(EP + base, CE)], ae1_v)

    def grp(gi, _):
        s = src_v[pl.ds(gi * 16, 16)]
        d = dst_v[pl.ds(gi * 16, 16)]
        e0 = ae0_v[pl.ds(gi * 16, 16)]
        e1 = ae1_v[pl.ds(gi * 16, 16)]
        x0 = jnp.exp(_lrelu(plsc.load_gather(as0, [s])
                            + plsc.load_gather(ad0, [d]) + e0))
        x1 = jnp.exp(_lrelu(plsc.load_gather(as1, [s])
                            + plsc.load_gather(ad1, [d]) + e1))
        w0_v[pl.ds(gi * 16, 16)] = x0 / (plsc.load_gather(dn0, [d]) + 1e-16)
        w1_v[pl.ds(gi * 16, 16)] = x1 / (plsc.load_gather(dn1, [d]) + 1e-16)
        return 0

    lax.fori_loop(0, CE // 16, grp, 0)
    pltpu.sync_copy(w0_v, wf.at[pl.ds(base, CE)])
    pltpu.sync_copy(w1_v, wf.at[pl.ds(EP + base, CE)])


HC = H * C


def _k2_body(srcm, dstm, wf, xhf, swf, aggf,
             src_v, dst_v, w0_v, w1_v,
             fsrc, fdl, fw0, fw1,
             rows, rows2, sw0_v, sw1_v, acc, sem, sem2):
    wid = _wid()
    lane = lax.iota(I32, 16)
    z16f = jnp.zeros((16,), F32)
    z16i = jnp.zeros((16,), I32)

    def add_row(rows_ref, dl_s, w0s, w1s, j):
        for c in range(C // 16):
            r0 = rows_ref[pl.ds(j * HC + c * 16, 16)]
            r1 = rows_ref[pl.ds(j * HC + C + c * 16, 16)]
            o = dl_s * C + c * 16
            acc[pl.ds(o, 16)] = acc[pl.ds(o, 16)] + w0s * r0 + w1s * r1

    def fire_batch(b, rows_ref, s):
        def fire(j, _):
            p = b * 8 + j
            sidx = fsrc[pl.ds(p, 16)][0]
            pltpu.async_copy(xhf.at[pl.ds(sidx * HC, HC)],
                             rows_ref.at[pl.ds(j * HC, HC)], s)
            return 0
        lax.fori_loop(0, 8, fire, 0)

    def proc_batch(b, rows_ref, s):
        pltpu.make_async_copy(xhf.at[pl.ds(0, 8 * HC)],
                              rows_ref.at[pl.ds(0, 8 * HC)], s).wait()
        for j in range(8):
            p = b * 8 + j
            dl_s = fdl[pl.ds(p, 16)][0]
            w0s = fw0[pl.ds(p, 16)][0]
            w1s = fw1[pl.ds(p, 16)][0]
            add_row(rows_ref, dl_s, w0s, w1s, j)

    def rangepass(rp, _):
        lo = wid * RN + rp * (NT * RN)

        def zr(r, _):
            acc[pl.ds(r * 16, 16)] = z16f
            return 0
        lax.fori_loop(0, ((RN + 1) * C) // 16, zr, 0)

        def chunk(ci, _):
            base = ci * CE
            pltpu.sync_copy(srcm.at[pl.ds(base, CE)], src_v)
            pltpu.sync_copy(dstm.at[pl.ds(base, CE)], dst_v)
            pltpu.sync_copy(wf.at[pl.ds(base, CE)], w0_v)
            pltpu.sync_copy(wf.at[pl.ds(EP + base, CE)], w1_v)

            def grp(gi, off):
                s = src_v[pl.ds(gi * 16, 16)]
                d = dst_v[pl.ds(gi * 16, 16)]
                w0 = w0_v[pl.ds(gi * 16, 16)]
                w1 = w1_v[pl.ds(gi * 16, 16)]
                dl = d - lo
                m = (dl >= 0) & (dl < RN)
                mi = jnp.where(m, jnp.ones((16,), I32), z16i)
                cs = plsc.cumsum(mi)
                pos = jnp.maximum(off + cs - 1, 0)
                plsc.store_scatter(fsrc, [pos], s, mask=m)
                plsc.store_scatter(fdl, [pos], dl, mask=m)
                plsc.store_scatter(fw0, [pos], w0, mask=m)
                plsc.store_scatter(fw1, [pos], w1, mask=m)
                return off + jnp.sum(mi)

            off = lax.fori_loop(0, CE // 16, grp, jnp.int32(0))
            padidx = off + lane
            plsc.store_scatter(fsrc, [padidx], z16i)
            plsc.store_scatter(fdl, [padidx], jnp.full((16,), RN, I32))
            plsc.store_scatter(fw0, [padidx], z16f)
            plsc.store_scatter(fw1, [padidx], z16f)
            nb = lax.div(off + 7, 8)

            @pl.when(nb > 0)
            def _():
                fire_batch(0, rows, sem)

            def bat(bi, _):
                even = (bi % 2) == 0
                more = bi + 1 < nb

                @pl.when(more & even)
                def _():
                    fire_batch(bi + 1, rows2, sem2)

                @pl.when(more & (~even))
                def _():
                    fire_batch(bi + 1, rows, sem)

                @pl.when(even)
                def _():
                    proc_batch(bi, rows, sem)

                @pl.when(~even)
                def _():
                    proc_batch(bi, rows2, sem2)
                return 0

            lax.fori_loop(0, nb, bat, 0)
            return 0

        lax.fori_loop(0, NT, chunk, 0)

        pltpu.sync_copy(swf.at[pl.ds(lo, RN)], sw0_v.at[pl.ds(0, RN)])
        pltpu.sync_copy(swf.at[pl.ds(NP + lo, RN)], sw1_v.at[pl.ds(0, RN)])

        def sb(k, _):
            pltpu.sync_copy(xhf.at[pl.ds((lo + k * 8) * HC, 8 * HC)], rows)

            def rowj2(j, _):
                nl = k * 8 + j
                w0s = sw0_v[pl.ds(nl, 16)][0]
                w1s = sw1_v[pl.ds(nl, 16)][0]
                add_row(rows, nl, w0s, w1s, j)
                return 0
            lax.fori_loop(0, 8, rowj2, 0)
            return 0

        lax.fori_loop(0, RN // 8, sb, 0)
        pltpu.sync_copy(acc.at[pl.ds(0, RN * C)], aggf.at[pl.ds(lo * C, RN * C)])
        return 0

    lax.fori_loop(0, NPASS, rangepass, 0)


def kernel(x, edge_index, edge_attr, batch, lin_W, att_src, att_dst, att_edge,
           lin_edge_W, gat_bias, gn_weight, gn_bias, gn_mean_scale, prelu_a,
           res_W, res_b, g1_W, g1_b, pa1, g2_W, g2_b, pa2, g3_W, g3_b):
    f32 = F32
    xp = jnp.pad(x.astype(f32), ((0, NP - N), (0, 0)))
    eap = jnp.pad(edge_attr.astype(f32), ((0, EP - E), (0, 0)))
    srcp = jnp.pad(edge_index[0].astype(I32), (0, EP - E))
    dstp = jnp.pad(edge_index[1].astype(I32), (0, EP - E),
                   constant_values=NP - 1)
    batchp = jnp.pad(batch.astype(I32), (0, NP - N),
                     constant_values=G).reshape(NB, 1, 256)
    att_s8 = jnp.zeros((8, C), f32).at[:H].set(att_src.astype(f32))
    att_d8 = jnp.zeros((8, C), f32).at[:H].set(att_dst.astype(f32))
    att_e8 = jnp.zeros((8, C), f32).at[:H].set(att_edge.astype(f32))
    bias1 = gat_bias.reshape(1, C).astype(f32)
    gnw1 = gn_weight.reshape(1, C).astype(f32)
    gnb1 = gn_bias.reshape(1, C).astype(f32)
    gms1 = gn_mean_scale.reshape(1, C).astype(f32)
    resb1 = res_b.reshape(1, C).astype(f32)
    g1b1 = g1_b.reshape(1, C).astype(f32)
    g2b1 = g2_b.reshape(1, C).astype(f32)
    pa_s = prelu_a.reshape(1, 1).astype(f32)
    pa1_s = pa1.reshape(1, 1).astype(f32)
    pa2_s = pa2.reshape(1, 1).astype(f32)
    g3b_s = g3_b.reshape(1, 1).astype(f32)

    xh, a8 = pl.pallas_call(
        _tc_pre_body,
        grid=(NB,),
        in_specs=[
            pl.BlockSpec((256, D), lambda i: (i, 0)),
            pl.BlockSpec((D, H * C), lambda i: (0, 0)),
            pl.BlockSpec((8, C), lambda i: (0, 0)),
            pl.BlockSpec((8, C), lambda i: (0, 0)),
        ],
        out_specs=[
            pl.BlockSpec((256, H * C), lambda i: (i, 0)),
            pl.BlockSpec((8, 256), lambda i: (0, i)),
        ],
        out_shape=[
            jax.ShapeDtypeStruct((NP, H * C), f32),
            jax.ShapeDtypeStruct((8, NP), f32),
        ],
        scratch_shapes=[pltpu.VMEM((8, 128), f32)],
    )(xp, lin_W.astype(f32), att_s8, att_d8)

    ae8 = pl.pallas_call(
        _tc_ae_body,
        grid=(EB,),
        in_specs=[
            pl.BlockSpec((512, D), lambda i: (i, 0)),
            pl.BlockSpec((D, H * C), lambda i: (0, 0)),
            pl.BlockSpec((8, C), lambda i: (0, 0)),
        ],
        out_specs=pl.BlockSpec((8, 512), lambda i: (0, i)),
        out_shape=jax.ShapeDtypeStruct((8, EP), f32),
        scratch_shapes=[pltpu.VMEM((8, 128), f32)],
    )(eap, lin_edge_W.astype(f32), att_e8)

    mesh = plsc.VectorSubcoreMesh(core_axis_name="c", subcore_axis_name="s")
    a4f = a8.reshape(8 * NP)[:4 * NP]
    aef = ae8.reshape(8 * EP)[:2 * EP]
    xhf = xh.reshape(NP * H * C)

    denf, swf = pl.kernel(
        _k1_body,
        out_type=(jax.ShapeDtypeStruct((2 * NP,), f32),
                  jax.ShapeDtypeStruct((2 * NP,), f32)),
        mesh=mesh,
        compiler_params=pltpu.CompilerParams(needs_layout_passes=False),
        scratch_types=[
            pltpu.VMEM((NP,), f32), pltpu.VMEM((NP,), f32),
            pltpu.VMEM((NP,), f32), pltpu.VMEM((NP,), f32),
            pltpu.VMEM((16 * R1,), f32), pltpu.VMEM((16 * R1,), f32),
            pltpu.VMEM((16 * R1,), f32), pltpu.VMEM((16 * R1,), f32),
            pltpu.VMEM((16 * R1,), f32),
            pltpu.VMEM((CE,), I32), pltpu.VMEM((CE,), I32),
            pltpu.VMEM((CE,), f32), pltpu.VMEM((CE,), f32),
            pltpu.VMEM((R1,), f32), pltpu.VMEM((R1,), f32),
            pltpu.VMEM((R1,), f32), pltpu.VMEM((R1,), f32),
        ],
    )(a4f, aef, srcp, dstp)

    wf = pl.kernel(
        _k1b_body,
        out_type=jax.ShapeDtypeStruct((2 * EP,), f32),
        mesh=mesh,
        compiler_params=pltpu.CompilerParams(needs_layout_passes=False),
        scratch_types=[
            pltpu.VMEM((NP,), f32), pltpu.VMEM((NP,), f32),
            pltpu.VMEM((NP,), f32), pltpu.VMEM((NP,), f32),
            pltpu.VMEM((NP,), f32), pltpu.VMEM((NP,), f32),
            pltpu.VMEM((CE,), I32), pltpu.VMEM((CE,), I32),
            pltpu.VMEM((CE,), f32), pltpu.VMEM((CE,), f32),
            pltpu.VMEM((CE,), f32), pltpu.VMEM((CE,), f32),
        ],
    )(a4f, aef, denf, srcp, dstp)

    aggf = pl.kernel(
        _k2_body,
        out_type=jax.ShapeDtypeStruct((NP * C,), f32),
        mesh=mesh,
        compiler_params=pltpu.CompilerParams(needs_layout_passes=False),
        scratch_types=[
            pltpu.VMEM((CE,), I32), pltpu.VMEM((CE,), I32),
            pltpu.VMEM((CE,), f32), pltpu.VMEM((CE,), f32),
            pltpu.VMEM((CE + 48,), I32), pltpu.VMEM((CE + 48,), I32),
            pltpu.VMEM((CE + 48,), f32), pltpu.VMEM((CE + 48,), f32),
            pltpu.VMEM((8 * H * C,), f32), pltpu.VMEM((8 * H * C,), f32),
            pltpu.VMEM((RN + 16,), f32), pltpu.VMEM((RN + 16,), f32),
            pltpu.VMEM(((RN + 1) * C,), f32),
            pltpu.SemaphoreType.DMA, pltpu.SemaphoreType.DMA,
        ],
    )(srcp, dstp, wf, xhf, swf)
    agg = aggf.reshape(NP, C)

    s1, s2, cnt8 = pl.pallas_call(
        _tc_b1_body,
        grid=(NB,),
        in_specs=[
            pl.BlockSpec((256, C), lambda i: (i, 0)),
            pl.BlockSpec((1, C), lambda i: (0, 0)),
            pl.BlockSpec((1, 1, 256), lambda i: (i, 0, 0)),
        ],
        out_specs=[
            pl.BlockSpec((G, C), lambda i: (0, 0)),
            pl.BlockSpec((G, C), lambda i: (0, 0)),
            pl.BlockSpec((G, 128), lambda i: (0, 0)),
        ],
        out_shape=[
            jax.ShapeDtypeStruct((G, C), f32),
            jax.ShapeDtypeStruct((G, C), f32),
            jax.ShapeDtypeStruct((G, 128), f32),
        ],
    )(agg, bias1, batchp)

    h3, gate3, gmax8 = pl.pallas_call(
        _tc_b2_body,
        grid=(NB,),
        in_specs=[
            pl.BlockSpec((256, C), lambda i: (i, 0)),
            pl.BlockSpec((1, C), lambda i: (0, 0)),
            pl.BlockSpec((1, 1, 256), lambda i: (i, 0, 0)),
            pl.BlockSpec((256, D), lambda i: (i, 0)),
            pl.BlockSpec((G, C), lambda i: (0, 0)),
            pl.BlockSpec((G, C), lambda i: (0, 0)),
            pl.BlockSpec((G, 128), lambda i: (0, 0)),
            pl.BlockSpec((1, C), lambda i: (0, 0)),
            pl.BlockSpec((1, C), lambda i: (0, 0)),
            pl.BlockSpec((1, C), lambda i: (0, 0)),
            pl.BlockSpec((D, C), lambda i: (0, 0)),
            pl.BlockSpec((1, C), lambda i: (0, 0)),
            pl.BlockSpec((1, 1), lambda i: (0, 0)),
            pl.BlockSpec((C, C), lambda i: (0, 0)),
            pl.BlockSpec((1, C), lambda i: (0, 0)),
            pl.BlockSpec((1, 1), lambda i: (0, 0)),
            pl.BlockSpec((C, C), lambda i: (0, 0)),
            pl.BlockSpec((1, C), lambda i: (0, 0)),
            pl.BlockSpec((1, 1), lambda i: (0, 0)),
            pl.BlockSpec((C, 1), lambda i: (0, 0)),
            pl.BlockSpec((1, 1), lambda i: (0, 0)),
        ],
        out_specs=[
            pl.BlockSpec((256, C), lambda i: (i, 0)),
            pl.BlockSpec((1, 1, 256), lambda i: (i, 0, 0)),
            pl.BlockSpec((G, 128), lambda i: (0, 0)),
        ],
        out_shape=[
            jax.ShapeDtypeStruct((NP, C), f32),
            jax.ShapeDtypeStruct((NB, 1, 256), f32),
            jax.ShapeDtypeStruct((G, 128), f32),
        ],
        scratch_shapes=[pltpu.VMEM((G, C), f32), pltpu.VMEM((G, C), f32)],
    )(agg, bias1, batchp, xp, s1, s2, cnt8, gnw1, gnb1, gms1,
      res_W.astype(f32), resb1, pa_s, g1_W.astype(f32), g1b1, pa1_s,
      g2_W.astype(f32), g2b1, pa2_s, g3_W.astype(f32), g3b_s)

    pooled = pl.pallas_call(
        _tc_b3_body,
        grid=(NB,),
        in_specs=[
            pl.BlockSpec((256, C), lambda i: (i, 0)),
            pl.BlockSpec((1, 1, 256), lambda i: (i, 0, 0)),
            pl.BlockSpec((G, 128), lambda i: (0, 0)),
            pl.BlockSpec((1, 1, 256), lambda i: (i, 0, 0)),
        ],
        out_specs=pl.BlockSpec((G, C), lambda i: (0, 0)),
        out_shape=jax.ShapeDtypeStruct((G, C), f32),
        scratch_shapes=[pltpu.VMEM((G, C), f32), pltpu.VMEM((G, 128), f32)],
    )(h3, gate3, gmax8, batchp)

    return pooled.reshape(G, 1, C)


# speedup vs baseline: 1.5066x; 1.5066x over previous
import functools

import jax
import jax.numpy as jnp
from jax import lax
from jax.experimental import pallas as pl
from jax.experimental.pallas import tpu as pltpu
from jax.experimental.pallas import tpu_sc as plsc

N = 10000
E = 160000
D = 128
H = 2
C = 768
G = 8

NT = 32
NP = 10240
EP = 160256
CE = EP // NT
R1 = NP // NT
RN = 80
NPASS = NP // (NT * RN)
NB = NP // 256
EB = EP // 512
F32 = jnp.float32
I32 = jnp.int32


def _tc_pre_body(x_ref, lw_ref, atts_ref, attd_ref, xh_ref, a8_ref, w_scr):
    i = pl.program_id(0)

    @pl.when(i == 0)
    def _():
        lw = lw_ref[...]
        asf = atts_ref[...]
        adf = attd_ref[...]
        parts = []
        for tab in (asf, adf):
            for h in range(H):
                parts.append(lax.dot_general(
                    tab[h:h + 1, :], lw[:, h * C:(h + 1) * C],
                    (((1,), (1,)), ((), ())), preferred_element_type=F32))
        parts.append(jnp.zeros((4, 128), F32))
        w_scr[...] = jnp.concatenate(parts, axis=0)

    xb = x_ref[...]
    xh_ref[...] = lax.dot_general(xb, lw_ref[...], (((1,), (0,)), ((), ())),
                                  preferred_element_type=F32)
    a8_ref[...] = lax.dot_general(w_scr[...], xb, (((1,), (1,)), ((), ())),
                                  preferred_element_type=F32)


def _tc_ae_body(e_ref, lw_ref, atte_ref, ae8_ref, w_scr):
    i = pl.program_id(0)

    @pl.when(i == 0)
    def _():
        lw = lw_ref[...]
        aef = atte_ref[...]
        parts = []
        for h in range(H):
            parts.append(lax.dot_general(
                aef[h:h + 1, :], lw[:, h * C:(h + 1) * C],
                (((1,), (1,)), ((), ())), preferred_element_type=F32))
        parts.append(jnp.zeros((6, 128), F32))
        w_scr[...] = jnp.concatenate(parts, axis=0)

    ae8_ref[...] = lax.dot_general(w_scr[...], e_ref[...],
                                   (((1,), (1,)), ((), ())),
                                   preferred_element_type=F32)


def _onehot_t(b3_ref):
    bv = b3_ref[0]
    iota8 = lax.broadcasted_iota(I32, (G, 1), 0)
    return (bv == iota8).astype(F32)


def _tc_b1_body(agg_ref, bias_ref, b3_ref, s1_ref, s2_ref, cnt_ref):
    i = pl.program_id(0)

    @pl.when(i == 0)
    def _():
        s1_ref[...] = jnp.zeros((G, C), F32)
        s2_ref[...] = jnp.zeros((G, C), F32)
        cnt_ref[...] = jnp.zeros((G, 128), F32)

    hp = 0.5 * agg_ref[...] + bias_ref[...]
    obt = _onehot_t(b3_ref)
    s1_ref[...] += lax.dot_general(obt, hp, (((1,), (0,)), ((), ())),
                                   preferred_element_type=F32)
    s2_ref[...] += lax.dot_general(obt, hp * hp, (((1,), (0,)), ((), ())),
                                   preferred_element_type=F32)
    cnt_ref[...] += lax.dot_general(obt, jnp.ones((256, 128), F32),
                                    (((1,), (0,)), ((), ())),
                                    preferred_element_type=F32)


def _tc_b2_body(agg_ref, bias_ref, b3_ref, x_ref, s1_ref, s2_ref, cnt_ref,
                gnw_ref, gnb_ref, gms_ref, resw_ref, resb_ref, pa_ref,
                g1w_ref, g1b_ref, pa1_ref, g2w_ref, g2b_ref, pa2_ref,
                g3w_ref, g3b_ref,
                h3_ref, gate3_ref, gmax_ref, a_scr, b_scr):
    i = pl.program_id(0)

    @pl.when(i == 0)
    def _():
        c = jnp.maximum(cnt_ref[...][:, 0:1], 1.0)
        s1 = s1_ref[...]
        mean = s1 / c
        mt = mean * gms_ref[...]
        var = s2_ref[...] / c - 2.0 * mt * (s1 / c) + mt * mt
        istd = lax.rsqrt(var + 1e-5)
        av = gnw_ref[...] * istd
        a_scr[...] = av
        b_scr[...] = gnb_ref[...] - mt * av
        gmax_ref[...] = jnp.full((G, 128), -1e30, F32)

    hp = 0.5 * agg_ref[...] + bias_ref[...]
    obt = _onehot_t(b3_ref)
    hn = hp * lax.dot_general(obt, a_scr[...], (((0,), (0,)), ((), ())),
                              preferred_element_type=F32)
    hn = hn + lax.dot_general(obt, b_scr[...], (((0,), (0,)), ((), ())),
                              preferred_element_type=F32)
    h3 = hn + lax.dot_general(x_ref[...], resw_ref[...],
                              (((1,), (0,)), ((), ())),
                              preferred_element_type=F32) + resb_ref[...]
    h3 = jnp.where(h3 >= 0.0, h3, pa_ref[...] * h3)
    h3_ref[...] = h3
    g = lax.dot_general(h3, g1w_ref[...], (((1,), (0,)), ((), ())),
                        preferred_element_type=F32) + g1b_ref[...]
    g = jnp.where(g >= 0.0, g, pa1_ref[...] * g)
    g = lax.dot_general(g, g2w_ref[...], (((1,), (0,)), ((), ())),
                        preferred_element_type=F32) + g2b_ref[...]
    g = jnp.where(g >= 0.0, g, pa2_ref[...] * g)
    gate_row = lax.dot_general(g3w_ref[...], g, (((0,), (1,)), ((), ())),
                               preferred_element_type=F32) + g3b_ref[...]
    gate3_ref[...] = gate_row.reshape(1, 1, 256)
    mg = jnp.where(obt > 0.0, gate_row, -1e30)
    gm = jnp.max(mg, axis=1, keepdims=True)
    gmax_ref[...] = jnp.maximum(gmax_ref[...], jnp.broadcast_to(gm, (G, 128)))


def _tc_b3_body(h3_ref, gate3_ref, gmax_ref, b3_ref, pooled_ref,
                sgeh_scr, gden_scr):
    i = pl.program_id(0)

    @pl.when(i == 0)
    def _():
        sgeh_scr[...] = jnp.zeros((G, C), F32)
        gden_scr[...] = jnp.zeros((G, 128), F32)

    obt = _onehot_t(b3_ref)
    gmv = gmax_ref[...][:, 0:1]
    gmv = jnp.where(gmv > -1e29, gmv, 0.0)
    gm_row = lax.dot_general(gmv, obt, (((0,), (0,)), ((), ())),
                             preferred_element_type=F32)
    ge_row = jnp.exp(gate3_ref[0] - gm_row)
    wob = obt * ge_row
    sgeh_scr[...] += lax.dot_general(wob, h3_ref[...], (((1,), (0,)), ((), ())),
                                     preferred_element_type=F32)
    gden_scr[...] += lax.dot_general(wob, jnp.ones((256, 128), F32),
                                     (((1,), (0,)), ((), ())),
                                     preferred_element_type=F32)
    pooled_ref[...] = sgeh_scr[...] / (gden_scr[...][:, 0:1] + 1e-16)


def _lrelu(v):
    return jnp.where(v >= 0.0, v, 0.2 * v)


def _wid():
    return lax.axis_index("c") * 16 + lax.axis_index("s")


def _zero_arr(ref, ngroups):
    z = jnp.zeros((16,), F32)

    def b(i, _):
        ref[pl.ds(i * 16, 16)] = z
        return 0
    lax.fori_loop(0, ngroups, b, 0)


def _k1_body(a4f, aef, srcm, dstm, denf, swf,
             as0, as1, ad0, ad1,
             d0a, d1a, sa0a, sa1a, dga,
             src_v, dst_v, ae0_v, ae1_v,
             den0_o, den1_o, sw0_o, sw1_o):
    wid = _wid()
    lo = wid * R1
    pltpu.sync_copy(a4f.at[pl.ds(0 * NP, NP)], as0)
    pltpu.sync_copy(a4f.at[pl.ds(1 * NP, NP)], as1)
    pltpu.sync_copy(a4f.at[pl.ds(2 * NP, NP)], ad0)
    pltpu.sync_copy(a4f.at[pl.ds(3 * NP, NP)], ad1)
    lane = lax.iota(I32, 16)
    for r in (d0a, d1a, sa0a, sa1a, dga):
        _zero_arr(r, R1)
    ones16 = jnp.ones((16,), F32)

    def chunk(ci, _):
        base = ci * CE
        pltpu.sync_copy(srcm.at[pl.ds(base, CE)], src_v)
        pltpu.sync_copy(dstm.at[pl.ds(base, CE)], dst_v)
        pltpu.sync_copy(aef.at[pl.ds(base, CE)], ae0_v)
        pltpu.sync_copy(aef.at[pl.ds(EP + base, CE)], ae1_v)

        def grp(gi, _):
            s = src_v[pl.ds(gi * 16, 16)]
            d = dst_v[pl.ds(gi * 16, 16)]
            e0 = ae0_v[pl.ds(gi * 16, 16)]
            e1 = ae1_v[pl.ds(gi * 16, 16)]
            x0 = jnp.exp(_lrelu(plsc.load_gather(as0, [s])
                                + plsc.load_gather(ad0, [d]) + e0))
            x1 = jnp.exp(_lrelu(plsc.load_gather(as1, [s])
                                + plsc.load_gather(ad1, [d]) + e1))
            dl = d - lo
            m = (dl >= 0) & (dl < R1)
            idx = lane * R1 + jnp.where(m, dl, 0)
            plsc.addupdate_scatter(d0a, [idx], x0, mask=m)
            plsc.addupdate_scatter(d1a, [idx], x1, mask=m)
            plsc.addupdate_scatter(sa0a, [idx], e0, mask=m)
            plsc.addupdate_scatter(sa1a, [idx], e1, mask=m)
            plsc.addupdate_scatter(dga, [idx], ones16, mask=m)
            return 0

        lax.fori_loop(0, CE // 16, grp, 0)
        return 0

    lax.fori_loop(0, NT, chunk, 0)

    def fin(gi, _):
        off = gi * 16

        def red(ref):
            v = jnp.zeros((16,), F32)
            for l in range(16):
                v = v + ref[pl.ds(l * R1 + off, 16)]
            return v

        den0 = red(d0a)
        den1 = red(d1a)
        s0 = red(sa0a)
        s1v = red(sa1a)
        dg = jnp.maximum(red(dga), 1.0)
        el0 = jnp.exp(_lrelu(as0[pl.ds(lo + off, 16)]
                             + ad0[pl.ds(lo + off, 16)] + s0 / dg))
        el1 = jnp.exp(_lrelu(as1[pl.ds(lo + off, 16)]
                             + ad1[pl.ds(lo + off, 16)] + s1v / dg))
        t0 = den0 + el0
        t1 = den1 + el1
        den0_o[pl.ds(off, 16)] = t0
        den1_o[pl.ds(off, 16)] = t1
        sw0_o[pl.ds(off, 16)] = el0 / (t0 + 1e-16)
        sw1_o[pl.ds(off, 16)] = el1 / (t1 + 1e-16)
        return 0

    lax.fori_loop(0, R1 // 16, fin, 0)
    pltpu.sync_copy(den0_o, denf.at[pl.ds(lo, R1)])
    pltpu.sync_copy(den1_o, denf.at[pl.ds(NP + lo, R1)])
    pltpu.sync_copy(sw0_o, swf.at[pl.ds(lo, R1)])
    pltpu.sync_copy(sw1_o, swf.at[pl.ds(NP + lo, R1)])


def _k1b_body(a4f, aef, denf, srcm, dstm, wf,
              as0, as1, ad0, ad1, dn0, dn1,
              src_v, dst_v, ae0_v, ae1_v, w0_v, w1_v):
    wid = _wid()
    base = wid * CE
    pltpu.sync_copy(a4f.at[pl.ds(0 * NP, NP)], as0)
    pltpu.sync_copy(a4f.at[pl.ds(1 * NP, NP)], as1)
    pltpu.sync_copy(a4f.at[pl.ds(2 * NP, NP)], ad0)
    pltpu.sync_copy(a4f.at[pl.ds(3 * NP, NP)], ad1)
    pltpu.sync_copy(denf.at[pl.ds(0 * NP, NP)], dn0)
    pltpu.sync_copy(denf.at[pl.ds(1 * NP, NP)], dn1)
    pltpu.sync_copy(srcm.at[pl.ds(base, CE)], src_v)
    pltpu.sync_copy(dstm.at[pl.ds(base, CE)], dst_v)
    pltpu.sync_copy(aef.at[pl.ds(base, CE)], ae0_v)
    pltpu.sync_copy(aef.at[pl.ds(EP + base, CE)], ae1_v)

    def grp(gi, _):
        s = src_v[pl.ds(gi * 16, 16)]
        d = dst_v[pl.ds(gi * 16, 16)]
        e0 = ae0_v[pl.ds(gi * 16, 16)]
        e1 = ae1_v[pl.ds(gi * 16, 16)]
        x0 = jnp.exp(_lrelu(plsc.load_gather(as0, [s])
                            + plsc.load_gather(ad0, [d]) + e0))
        x1 = jnp.exp(_lrelu(plsc.load_gather(as1, [s])
                            + plsc.load_gather(ad1, [d]) + e1))
        w0_v[pl.ds(gi * 16, 16)] = x0 / (plsc.load_gather(dn0, [d]) + 1e-16)
        w1_v[pl.ds(gi * 16, 16)] = x1 / (plsc.load_gather(dn1, [d]) + 1e-16)
        return 0

    lax.fori_loop(0, CE // 16, grp, 0)
    pltpu.sync_copy(w0_v, wf.at[pl.ds(base, CE)])
    pltpu.sync_copy(w1_v, wf.at[pl.ds(EP + base, CE)])


HC = H * C


def _k2_body(srcm, dstm, wf, xhf, swf, aggf,
             src_v, dst_v, w0_v, w1_v,
             fsrc, fdl, fw0, fw1,
             rows, rows2, sw0_v, sw1_v, acc, sem, sem2):
    wid = _wid()
    lane = lax.iota(I32, 16)
    z16f = jnp.zeros((16,), F32)
    z16i = jnp.zeros((16,), I32)

    def add_row(rows_ref, dl_s, w0s, w1s, j):
        for c in range(C // 16):
            r0 = rows_ref[pl.ds(j * HC + c * 16, 16)]
            r1 = rows_ref[pl.ds(j * HC + C + c * 16, 16)]
            o = dl_s * C + c * 16
            acc[pl.ds(o, 16)] = acc[pl.ds(o, 16)] + w0s * r0 + w1s * r1

    def fire_batch(b, rows_ref, s):
        def fire(j, _):
            p = b * 8 + j
            sidx = fsrc[pl.ds(p, 16)][0]
            pltpu.async_copy(xhf.at[pl.ds(sidx * HC, HC)],
                             rows_ref.at[pl.ds(j * HC, HC)], s)
            return 0
        lax.fori_loop(0, 8, fire, 0)

    def proc_batch(b, rows_ref, s):
        pltpu.make_async_copy(xhf.at[pl.ds(0, 8 * HC)],
                              rows_ref.at[pl.ds(0, 8 * HC)], s).wait()

        def rowj(j, _):
            p = b * 8 + j
            dl_s = fdl[pl.ds(p, 16)][0]
            w0s = fw0[pl.ds(p, 16)][0]
            w1s = fw1[pl.ds(p, 16)][0]
            add_row(rows_ref, dl_s, w0s, w1s, j)
            return 0
        lax.fori_loop(0, 8, rowj, 0)

    def rangepass(rp, _):
        lo = wid * RN + rp * (NT * RN)

        def zr(r, _):
            acc[pl.ds(r * 16, 16)] = z16f
            return 0
        lax.fori_loop(0, ((RN + 1) * C) // 16, zr, 0)

        def chunk(ci, _):
            base = ci * CE
            pltpu.sync_copy(srcm.at[pl.ds(base, CE)], src_v)
            pltpu.sync_copy(dstm.at[pl.ds(base, CE)], dst_v)
            pltpu.sync_copy(wf.at[pl.ds(base, CE)], w0_v)
            pltpu.sync_copy(wf.at[pl.ds(EP + base, CE)], w1_v)

            def grp(gi, off):
                s = src_v[pl.ds(gi * 16, 16)]
                d = dst_v[pl.ds(gi * 16, 16)]
                w0 = w0_v[pl.ds(gi * 16, 16)]
                w1 = w1_v[pl.ds(gi * 16, 16)]
                dl = d - lo
                m = (dl >= 0) & (dl < RN)
                mi = jnp.where(m, jnp.ones((16,), I32), z16i)
                cs = plsc.cumsum(mi)
                pos = jnp.maximum(off + cs - 1, 0)
                plsc.store_scatter(fsrc, [pos], s, mask=m)
                plsc.store_scatter(fdl, [pos], dl, mask=m)
                plsc.store_scatter(fw0, [pos], w0, mask=m)
                plsc.store_scatter(fw1, [pos], w1, mask=m)
                return off + jnp.sum(mi)

            off = lax.fori_loop(0, CE // 16, grp, jnp.int32(0))
            padidx = off + lane
            plsc.store_scatter(fsrc, [padidx], z16i)
            plsc.store_scatter(fdl, [padidx], jnp.full((16,), RN, I32))
            plsc.store_scatter(fw0, [padidx], z16f)
            plsc.store_scatter(fw1, [padidx], z16f)
            nb = lax.div(off + 7, 8)

            @pl.when(nb > 0)
            def _():
                fire_batch(0, rows, sem)

            def bat(bi, _):
                even = (bi % 2) == 0
                more = bi + 1 < nb

                @pl.when(more & even)
                def _():
                    fire_batch(bi + 1, rows2, sem2)

                @pl.when(more & (~even))
                def _():
                    fire_batch(bi + 1, rows, sem)

                @pl.when(even)
                def _():
                    proc_batch(bi, rows, sem)

                @pl.when(~even)
                def _():
                    proc_batch(bi, rows2, sem2)
                return 0

            lax.fori_loop(0, nb, bat, 0)
            return 0

        lax.fori_loop(0, NT, chunk, 0)

        pltpu.sync_copy(swf.at[pl.ds(lo, RN)], sw0_v.at[pl.ds(0, RN)])
        pltpu.sync_copy(swf.at[pl.ds(NP + lo, RN)], sw1_v.at[pl.ds(0, RN)])

        def sb(k, _):
            pltpu.sync_copy(xhf.at[pl.ds((lo + k * 8) * HC, 8 * HC)], rows)

            def rowj2(j, _):
                nl = k * 8 + j
                w0s = sw0_v[pl.ds(nl, 16)][0]
                w1s = sw1_v[pl.ds(nl, 16)][0]
                add_row(rows, nl, w0s, w1s, j)
                return 0
            lax.fori_loop(0, 8, rowj2, 0)
            return 0

        lax.fori_loop(0, RN // 8, sb, 0)
        pltpu.sync_copy(acc.at[pl.ds(0, RN * C)], aggf.at[pl.ds(lo * C, RN * C)])
        return 0

    lax.fori_loop(0, NPASS, rangepass, 0)


def kernel(x, edge_index, edge_attr, batch, lin_W, att_src, att_dst, att_edge,
           lin_edge_W, gat_bias, gn_weight, gn_bias, gn_mean_scale, prelu_a,
           res_W, res_b, g1_W, g1_b, pa1, g2_W, g2_b, pa2, g3_W, g3_b):
    f32 = F32
    xp = jnp.pad(x.astype(f32), ((0, NP - N), (0, 0)))
    eap = jnp.pad(edge_attr.astype(f32), ((0, EP - E), (0, 0)))
    srcp = jnp.pad(edge_index[0].astype(I32), (0, EP - E))
    dstp = jnp.pad(edge_index[1].astype(I32), (0, EP - E),
                   constant_values=NP - 1)
    batchp = jnp.pad(batch.astype(I32), (0, NP - N),
                     constant_values=G).reshape(NB, 1, 256)
    att_s8 = jnp.zeros((8, C), f32).at[:H].set(att_src.astype(f32))
    att_d8 = jnp.zeros((8, C), f32).at[:H].set(att_dst.astype(f32))
    att_e8 = jnp.zeros((8, C), f32).at[:H].set(att_edge.astype(f32))
    bias1 = gat_bias.reshape(1, C).astype(f32)
    gnw1 = gn_weight.reshape(1, C).astype(f32)
    gnb1 = gn_bias.reshape(1, C).astype(f32)
    gms1 = gn_mean_scale.reshape(1, C).astype(f32)
    resb1 = res_b.reshape(1, C).astype(f32)
    g1b1 = g1_b.reshape(1, C).astype(f32)
    g2b1 = g2_b.reshape(1, C).astype(f32)
    pa_s = prelu_a.reshape(1, 1).astype(f32)
    pa1_s = pa1.reshape(1, 1).astype(f32)
    pa2_s = pa2.reshape(1, 1).astype(f32)
    g3b_s = g3_b.reshape(1, 1).astype(f32)

    xh, a8 = pl.pallas_call(
        _tc_pre_body,
        grid=(NB,),
        in_specs=[
            pl.BlockSpec((256, D), lambda i: (i, 0)),
            pl.BlockSpec((D, H * C), lambda i: (0, 0)),
            pl.BlockSpec((8, C), lambda i: (0, 0)),
            pl.BlockSpec((8, C), lambda i: (0, 0)),
        ],
        out_specs=[
            pl.BlockSpec((256, H * C), lambda i: (i, 0)),
            pl.BlockSpec((8, 256), lambda i: (0, i)),
        ],
        out_shape=[
            jax.ShapeDtypeStruct((NP, H * C), f32),
            jax.ShapeDtypeStruct((8, NP), f32),
        ],
        scratch_shapes=[pltpu.VMEM((8, 128), f32)],
    )(xp, lin_W.astype(f32), att_s8, att_d8)

    ae8 = pl.pallas_call(
        _tc_ae_body,
        grid=(EB,),
        in_specs=[
            pl.BlockSpec((512, D), lambda i: (i, 0)),
            pl.BlockSpec((D, H * C), lambda i: (0, 0)),
            pl.BlockSpec((8, C), lambda i: (0, 0)),
        ],
        out_specs=pl.BlockSpec((8, 512), lambda i: (0, i)),
        out_shape=jax.ShapeDtypeStruct((8, EP), f32),
        scratch_shapes=[pltpu.VMEM((8, 128), f32)],
    )(eap, lin_edge_W.astype(f32), att_e8)

    mesh = plsc.VectorSubcoreMesh(core_axis_name="c", subcore_axis_name="s")
    a4f = a8.reshape(8 * NP)[:4 * NP]
    aef = ae8.reshape(8 * EP)[:2 * EP]
    xhf = xh.reshape(NP * H * C)

    denf, swf = pl.kernel(
        _k1_body,
        out_type=(jax.ShapeDtypeStruct((2 * NP,), f32),
                  jax.ShapeDtypeStruct((2 * NP,), f32)),
        mesh=mesh,
        compiler_params=pltpu.CompilerParams(needs_layout_passes=False),
        scratch_types=[
            pltpu.VMEM((NP,), f32), pltpu.VMEM((NP,), f32),
            pltpu.VMEM((NP,), f32), pltpu.VMEM((NP,), f32),
            pltpu.VMEM((16 * R1,), f32), pltpu.VMEM((16 * R1,), f32),
            pltpu.VMEM((16 * R1,), f32), pltpu.VMEM((16 * R1,), f32),
            pltpu.VMEM((16 * R1,), f32),
            pltpu.VMEM((CE,), I32), pltpu.VMEM((CE,), I32),
            pltpu.VMEM((CE,), f32), pltpu.VMEM((CE,), f32),
            pltpu.VMEM((R1,), f32), pltpu.VMEM((R1,), f32),
            pltpu.VMEM((R1,), f32), pltpu.VMEM((R1,), f32),
        ],
    )(a4f, aef, srcp, dstp)

    wf = pl.kernel(
        _k1b_body,
        out_type=jax.ShapeDtypeStruct((2 * EP,), f32),
        mesh=mesh,
        compiler_params=pltpu.CompilerParams(needs_layout_passes=False),
        scratch_types=[
            pltpu.VMEM((NP,), f32), pltpu.VMEM((NP,), f32),
            pltpu.VMEM((NP,), f32), pltpu.VMEM((NP,), f32),
            pltpu.VMEM((NP,), f32), pltpu.VMEM((NP,), f32),
            pltpu.VMEM((CE,), I32), pltpu.VMEM((CE,), I32),
            pltpu.VMEM((CE,), f32), pltpu.VMEM((CE,), f32),
            pltpu.VMEM((CE,), f32), pltpu.VMEM((CE,), f32),
        ],
    )(a4f, aef, denf, srcp, dstp)

    aggf = pl.kernel(
        _k2_body,
        out_type=jax.ShapeDtypeStruct((NP * C,), f32),
        mesh=mesh,
        compiler_params=pltpu.CompilerParams(needs_layout_passes=False),
        scratch_types=[
            pltpu.VMEM((CE,), I32), pltpu.VMEM((CE,), I32),
            pltpu.VMEM((CE,), f32), pltpu.VMEM((CE,), f32),
            pltpu.VMEM((CE + 48,), I32), pltpu.VMEM((CE + 48,), I32),
            pltpu.VMEM((CE + 48,), f32), pltpu.VMEM((CE + 48,), f32),
            pltpu.VMEM((8 * H * C,), f32), pltpu.VMEM((8 * H * C,), f32),
            pltpu.VMEM((RN + 16,), f32), pltpu.VMEM((RN + 16,), f32),
            pltpu.VMEM(((RN + 1) * C,), f32),
            pltpu.SemaphoreType.DMA, pltpu.SemaphoreType.DMA,
        ],
    )(srcp, dstp, wf, xhf, swf)
    agg = aggf.reshape(NP, C)

    s1, s2, cnt8 = pl.pallas_call(
        _tc_b1_body,
        grid=(NB,),
        in_specs=[
            pl.BlockSpec((256, C), lambda i: (i, 0)),
            pl.BlockSpec((1, C), lambda i: (0, 0)),
            pl.BlockSpec((1, 1, 256), lambda i: (i, 0, 0)),
        ],
        out_specs=[
            pl.BlockSpec((G, C), lambda i: (0, 0)),
            pl.BlockSpec((G, C), lambda i: (0, 0)),
            pl.BlockSpec((G, 128), lambda i: (0, 0)),
        ],
        out_shape=[
            jax.ShapeDtypeStruct((G, C), f32),
            jax.ShapeDtypeStruct((G, C), f32),
            jax.ShapeDtypeStruct((G, 128), f32),
        ],
    )(agg, bias1, batchp)

    h3, gate3, gmax8 = pl.pallas_call(
        _tc_b2_body,
        grid=(NB,),
        in_specs=[
            pl.BlockSpec((256, C), lambda i: (i, 0)),
            pl.BlockSpec((1, C), lambda i: (0, 0)),
            pl.BlockSpec((1, 1, 256), lambda i: (i, 0, 0)),
            pl.BlockSpec((256, D), lambda i: (i, 0)),
            pl.BlockSpec((G, C), lambda i: (0, 0)),
            pl.BlockSpec((G, C), lambda i: (0, 0)),
            pl.BlockSpec((G, 128), lambda i: (0, 0)),
            pl.BlockSpec((1, C), lambda i: (0, 0)),
            pl.BlockSpec((1, C), lambda i: (0, 0)),
            pl.BlockSpec((1, C), lambda i: (0, 0)),
            pl.BlockSpec((D, C), lambda i: (0, 0)),
            pl.BlockSpec((1, C), lambda i: (0, 0)),
            pl.BlockSpec((1, 1), lambda i: (0, 0)),
            pl.BlockSpec((C, C), lambda i: (0, 0)),
            pl.BlockSpec((1, C), lambda i: (0, 0)),
            pl.BlockSpec((1, 1), lambda i: (0, 0)),
            pl.BlockSpec((C, C), lambda i: (0, 0)),
            pl.BlockSpec((1, C), lambda i: (0, 0)),
            pl.BlockSpec((1, 1), lambda i: (0, 0)),
            pl.BlockSpec((C, 1), lambda i: (0, 0)),
            pl.BlockSpec((1, 1), lambda i: (0, 0)),
        ],
        out_specs=[
            pl.BlockSpec((256, C), lambda i: (i, 0)),
            pl.BlockSpec((1, 1, 256), lambda i: (i, 0, 0)),
            pl.BlockSpec((G, 128), lambda i: (0, 0)),
        ],
        out_shape=[
            jax.ShapeDtypeStruct((NP, C), f32),
            jax.ShapeDtypeStruct((NB, 1, 256), f32),
            jax.ShapeDtypeStruct((G, 128), f32),
        ],
        scratch_shapes=[pltpu.VMEM((G, C), f32), pltpu.VMEM((G, C), f32)],
    )(agg, bias1, batchp, xp, s1, s2, cnt8, gnw1, gnb1, gms1,
      res_W.astype(f32), resb1, pa_s, g1_W.astype(f32), g1b1, pa1_s,
      g2_W.astype(f32), g2b1, pa2_s, g3_W.astype(f32), g3b_s)

    pooled = pl.pallas_call(
        _tc_b3_body,
        grid=(NB,),
        in_specs=[
            pl.BlockSpec((256, C), lambda i: (i, 0)),
            pl.BlockSpec((1, 1, 256), lambda i: (i, 0, 0)),
            pl.BlockSpec((G, 128), lambda i: (0, 0)),
            pl.BlockSpec((1, 1, 256), lambda i: (i, 0, 0)),
        ],
        out_specs=pl.BlockSpec((G, C), lambda i: (0, 0)),
        out_shape=jax.ShapeDtypeStruct((G, C), f32),
        scratch_shapes=[pltpu.VMEM((G, C), f32), pltpu.VMEM((G, 128), f32)],
    )(h3, gate3, gmax8, batchp)

    return pooled.reshape(G, 1, C)
